# baseline (speedup 1.0000x reference)
import functools

import jax
import jax.numpy as jnp
from jax import lax
from jax.experimental import pallas as pl
from jax.experimental.pallas import tpu as pltpu
from jax.experimental.pallas import tpu_sc as plsc

EPS = 1e-7

NCORES = 2
NSUB = 16
NTILES = NCORES * NSUB
CH = 128
NBUF = 4
LOOK = 2


def _mm_body(x_ref, w_ref, o_ref):
    res = jnp.dot(x_ref[...], w_ref[...],
                  preferred_element_type=jnp.float32)
    half = res.shape[1] // 2
    o_ref[0] = res[:, :half]
    o_ref[1] = res[:, half:]


def _matmul_split(x, w):
    n, din = x.shape
    dout = w.shape[1]
    bn = 1000
    return pl.pallas_call(
        _mm_body,
        grid=(n // bn,),
        in_specs=[
            pl.BlockSpec((bn, din), lambda i: (i, 0)),
            pl.BlockSpec((din, dout), lambda i: (0, 0)),
        ],
        out_specs=pl.BlockSpec((2, bn, dout // 2), lambda i: (0, i, 0)),
        out_shape=jax.ShapeDtypeStruct((2, n, dout // 2), jnp.float32),
    )(x, w)


def _rnorm(u):
    return jnp.maximum(jnp.sqrt(jnp.sum(u * u, axis=-1, keepdims=True)), EPS)


def _expmap0(u, sc):
    n = _rnorm(u)
    return jnp.tanh(sc * n) * u / (sc * n)


def _proj(p, sc):
    maxnorm = (1.0 - 1e-5) / sc
    n = _rnorm(p)
    return jnp.where(n > maxnorm, p / n * maxnorm, p)


def _logmap0(p, sc):
    n = _rnorm(p)
    arg = jnp.minimum(sc * n, 1.0 - 1e-5)
    atanh = 0.5 * jnp.log((1.0 + arg) / (1.0 - arg))
    return atanh * p / (sc * n)


def _mobius_add(x, y, c):
    x2 = jnp.sum(x * x, -1, keepdims=True)
    y2 = jnp.sum(y * y, -1, keepdims=True)
    xy = jnp.sum(x * y, -1, keepdims=True)
    num = (1 + 2 * c * xy + c * y2) * x + (1 - c * x2) * y
    den = 1 + 2 * c * xy + c * c * x2 * y2
    return num / jnp.maximum(den, EPS)


def _hyp_body(parts_ref, b_ref, c_ref, *rest, relu, has_w, concat):
    if has_w:
        w_ref, o_ref = rest
    else:
        (o_ref,) = rest
    c = jnp.maximum(c_ref[0], 1e-4)
    sc = jnp.sqrt(c)
    if concat:
        agg = jnp.concatenate([parts_ref[0], parts_ref[1]], axis=-1)
    else:
        agg = parts_ref[0] + parts_ref[1]
    p = _proj(_expmap0(agg, sc), sc)
    bh = _proj(_expmap0(b_ref[...], sc), sc)
    p = _proj(_mobius_add(p, bh, c), sc)
    out = _logmap0(p, sc)
    if relu:
        out = jnp.maximum(out, 0.0)
    if has_w:
        o_ref[...] = jnp.dot(out, w_ref[...],
                             preferred_element_type=jnp.float32)
    else:
        o_ref[...] = out


def _hyp(parts, b, c, w_next, relu, concat=False):
    _, n, dpart = parts.shape
    dout = 2 * dpart if concat else dpart
    bn = n // 8
    has_w = w_next is not None
    dnext = w_next.shape[1] if has_w else dout
    in_specs = [
        pl.BlockSpec((2, bn, dpart), lambda i: (0, i, 0)),
        pl.BlockSpec((1, dout), lambda i: (0, 0)),
        pl.BlockSpec(memory_space=pltpu.SMEM),
    ]
    args = [parts, b.reshape(1, dout), jnp.reshape(c, (1,))]
    if has_w:
        in_specs.append(pl.BlockSpec((dout, dnext), lambda i: (0, 0)))
        args.append(w_next)
    return pl.pallas_call(
        functools.partial(_hyp_body, relu=relu, has_w=has_w, concat=concat),
        grid=(n // bn,),
        in_specs=in_specs,
        out_specs=pl.BlockSpec((bn, dnext), lambda i: (i, 0)),
        out_shape=jax.ShapeDtypeStruct((n, dnext), jnp.float32),
    )(*args)


def _seg_kernel_body(dout, nchunks, rows_per_tile, col_split,
                     h_hbm, e_hbm, out_hbm,
                     src_v, dst_v, w_v, rows_v, h_sh, agg_sh, gsem, ssem):
    cid = lax.axis_index("c")
    sid = lax.axis_index("s")
    wid = sid if col_split else cid * NSUB + sid

    hrows = h_hbm.shape[-2] // NSUB
    row0 = sid * rows_per_tile
    d1 = pltpu.async_copy(e_hbm.at[0, wid], src_v, gsem.at[0])
    d2 = pltpu.async_copy(e_hbm.at[1, wid], dst_v, gsem.at[1])
    d3 = pltpu.async_copy(e_hbm.at[2, wid], w_v, gsem.at[2])
    if col_split:
        d4 = pltpu.async_copy(h_hbm.at[cid, pl.ds(sid * hrows, hrows)],
                              h_sh.at[pl.ds(sid * hrows, hrows)], gsem.at[3])
    else:
        d4 = pltpu.async_copy(h_hbm.at[pl.ds(sid * hrows, hrows)],
                              h_sh.at[pl.ds(sid * hrows, hrows)], gsem.at[3])

    def zrow(r, c2):
        for db in range(dout // 16):
            rows_v[0, r, pl.ds(db * 16, 16)] = jnp.zeros((16,), jnp.float32)
        return c2

    lax.fori_loop(0, CH, zrow, 0, unroll=8)
    zd = []
    for k in range(rows_per_tile // CH):
        zd.append(pltpu.async_copy(
            rows_v.at[0], agg_sh.at[pl.ds(row0 + k * CH, CH)],
            ssem.at[k % NBUF]))
    rem = rows_per_tile % CH
    if rem:
        zd.append(pltpu.async_copy(
            rows_v.at[0, pl.ds(0, rem)],
            agg_sh.at[pl.ds(row0 + (rows_per_tile // CH) * CH, rem)],
            ssem.at[rows_per_tile // CH % NBUF]))
    d1.wait()
    d2.wait()
    d3.wait()
    d4.wait()
    for d in zd:
        d.wait()
    plsc.subcore_barrier()

    def g_start(b, j):
        pltpu.async_copy(h_sh.at[src_v.at[j]], rows_v.at[b], gsem.at[b])

    def g_wait(b):
        pltpu.make_async_copy(h_sh.at[src_v.at[0]], rows_v.at[b],
                              gsem.at[b]).wait()

    def s_start(b, j):
        pltpu.async_copy(rows_v.at[b], agg_sh.at[dst_v.at[j]], ssem.at[b],
                         add=True)

    def s_wait(b):
        pltpu.make_async_copy(rows_v.at[b], agg_sh.at[dst_v.at[0]],
                              ssem.at[b]).wait()

    def scale(b, j):
        for g in range(CH // 16):
            wvec = lax.bitcast_convert_type(w_v[j, pl.ds(g * 16, 16)],
                                            jnp.float32)

            def edge_body(e16, c2, g=g, wvec=wvec):
                wv = lax.gather(
                    wvec, jnp.full((16, 1), 0, jnp.int32) + e16,
                    lax.GatherDimensionNumbers(
                        offset_dims=(), collapsed_slice_dims=(0,),
                        start_index_map=(0,)),
                    slice_sizes=(1,),
                    mode=lax.GatherScatterMode.PROMISE_IN_BOUNDS)
                e = g * 16 + e16
                for db in range(dout // 16):
                    seg = rows_v[b, e, pl.ds(db * 16, 16)]
                    rows_v[b, e, pl.ds(db * 16, 16)] = seg * wv
                return c2

            lax.fori_loop(0, 16, edge_body, 0, unroll=8)

    for jj in range(LOOK):
        g_start(jj, jj)

    def q_body(q, carry):
        for b in range(NBUF):
            j = q * NBUF + b
            g_wait(b)
            scale(b, j)
            bn = (b + LOOK) % NBUF

            @pl.when(j >= NBUF - LOOK)
            def _():
                s_wait(bn)

            @pl.when(j + LOOK < nchunks)
            def _():
                g_start(bn, j + LOOK)

            s_start(b, j)
        return carry

    lax.fori_loop(0, nchunks // NBUF, q_body, 0)
    for jj in range(nchunks - NBUF + LOOK, nchunks):
        s_wait(jj % NBUF)
    plsc.subcore_barrier()
    pltpu.sync_copy(agg_sh.at[pl.ds(row0, rows_per_tile)],
                    out_hbm.at[cid, pl.ds(row0, rows_per_tile)])


def _seg(h, e3, n, col_split=False):
    dout = h.shape[-1]
    nchunks = e3.shape[2]
    rows_per_tile = n // NSUB
    mesh = plsc.VectorSubcoreMesh(core_axis_name="c", subcore_axis_name="s")
    kfn = pl.kernel(
        functools.partial(_seg_kernel_body, dout, nchunks, rows_per_tile,
                          col_split),
        out_type=jax.ShapeDtypeStruct((NCORES, n, dout), jnp.float32),
        mesh=mesh,
        scratch_types=[
            pltpu.VMEM((nchunks, CH), jnp.int32),
            pltpu.VMEM((nchunks, CH), jnp.int32),
            pltpu.VMEM((nchunks, CH), jnp.int32),
            pltpu.VMEM((NBUF, CH, dout), jnp.float32),
            pltpu.VMEM_SHARED((h.shape[-2], dout), jnp.float32),
            pltpu.VMEM_SHARED((n, dout), jnp.float32),
            pltpu.SemaphoreType.DMA((NBUF,)),
            pltpu.SemaphoreType.DMA((NBUF,)),
        ],
        compiler_params=pltpu.CompilerParams(use_tc_tiling_on_sc=False),
    )
    return kfn(h, e3)


def kernel(x, edge_index, edge_weight, W0, b0, c0, W1, b1, c1, W2, b2, c2):
    n = x.shape[0]
    e = edge_weight.shape[0]
    per_tile = -(-e // NTILES)
    nchunks = -(-(-(-per_tile // CH)) // NBUF) * NBUF
    total = NTILES * nchunks * CH
    pad = total - e

    srcf = jnp.pad(edge_index[0].astype(jnp.int32), (0, pad))
    dstf = jnp.pad(edge_index[1].astype(jnp.int32), (0, pad))
    wf = lax.bitcast_convert_type(
        jnp.pad(edge_weight.astype(jnp.float32), (0, pad)), jnp.int32)
    ef = jnp.stack([srcf, dstf, wf])
    e3 = ef.reshape(3, NTILES, nchunks, CH)
    e2 = ef.reshape(3, NSUB, 2 * nchunks, CH)

    n_pad = -(-n // (NSUB * 8)) * (NSUB * 8)

    hs = _matmul_split(x, W0)
    parts = _seg(hs, e2, n_pad, col_split=True)
    h = _hyp(parts, b0, c0, W1, relu=True, concat=True)
    parts = _seg(h, e3, n_pad)
    h = _hyp(parts, b1, c1, W2, relu=True)
    parts = _seg(h, e3, n_pad)
    return _hyp(parts, b2, c2, None, relu=False)[:n]


# speedup vs baseline: 1.0066x; 1.0066x over previous
import functools

import jax
import jax.numpy as jnp
from jax import lax
from jax.experimental import pallas as pl
from jax.experimental.pallas import tpu as pltpu
from jax.experimental.pallas import tpu_sc as plsc

EPS = 1e-7

NCORES = 2
NSUB = 16
NTILES = NCORES * NSUB
CH = 128
NBUF = 4
LOOK = 2


def _mm_body(x_ref, w_ref, o_ref):
    res = jnp.dot(x_ref[...], w_ref[...],
                  preferred_element_type=jnp.float32)
    half = res.shape[1] // 2
    o_ref[0] = res[:, :half]
    o_ref[1] = res[:, half:]


def _matmul_split(x, w):
    n, din = x.shape
    dout = w.shape[1]
    bn = 1000
    return pl.pallas_call(
        _mm_body,
        grid=(n // bn,),
        in_specs=[
            pl.BlockSpec((bn, din), lambda i: (i, 0)),
            pl.BlockSpec((din, dout), lambda i: (0, 0)),
        ],
        out_specs=pl.BlockSpec((2, bn, dout // 2), lambda i: (0, i, 0)),
        out_shape=jax.ShapeDtypeStruct((2, n, dout // 2), jnp.float32),
    )(x, w)


def _rnorm(u):
    return jnp.maximum(jnp.sqrt(jnp.sum(u * u, axis=-1, keepdims=True)), EPS)


def _expmap0(u, sc):
    n = _rnorm(u)
    return jnp.tanh(sc * n) * u / (sc * n)


def _proj(p, sc):
    maxnorm = (1.0 - 1e-5) / sc
    n = _rnorm(p)
    return jnp.where(n > maxnorm, p / n * maxnorm, p)


def _logmap0(p, sc):
    n = _rnorm(p)
    arg = jnp.minimum(sc * n, 1.0 - 1e-5)
    atanh = 0.5 * jnp.log((1.0 + arg) / (1.0 - arg))
    return atanh * p / (sc * n)


def _mobius_add(x, y, c):
    x2 = jnp.sum(x * x, -1, keepdims=True)
    y2 = jnp.sum(y * y, -1, keepdims=True)
    xy = jnp.sum(x * y, -1, keepdims=True)
    num = (1 + 2 * c * xy + c * y2) * x + (1 - c * x2) * y
    den = 1 + 2 * c * xy + c * c * x2 * y2
    return num / jnp.maximum(den, EPS)


def _hyp_body(parts_ref, b_ref, c_ref, *rest, relu, has_w, concat):
    if has_w:
        w_ref, o_ref = rest
    else:
        (o_ref,) = rest
    c = jnp.maximum(c_ref[0], 1e-4)
    sc = jnp.sqrt(c)
    if concat:
        agg = jnp.concatenate([parts_ref[0], parts_ref[1]], axis=-1)
    else:
        agg = parts_ref[0] + parts_ref[1]
    p = _proj(_expmap0(agg, sc), sc)
    bh = _proj(_expmap0(b_ref[...], sc), sc)
    p = _proj(_mobius_add(p, bh, c), sc)
    out = _logmap0(p, sc)
    if relu:
        out = jnp.maximum(out, 0.0)
    if has_w:
        o_ref[...] = jnp.dot(out, w_ref[...],
                             preferred_element_type=jnp.float32)
    else:
        o_ref[...] = out


def _hyp(parts, b, c, w_next, relu, concat=False):
    _, n, dpart = parts.shape
    dout = 2 * dpart if concat else dpart
    bn = n // 8
    has_w = w_next is not None
    dnext = w_next.shape[1] if has_w else dout
    in_specs = [
        pl.BlockSpec((2, bn, dpart), lambda i: (0, i, 0)),
        pl.BlockSpec((1, dout), lambda i: (0, 0)),
        pl.BlockSpec(memory_space=pltpu.SMEM),
    ]
    args = [parts, b.reshape(1, dout), jnp.reshape(c, (1,))]
    if has_w:
        in_specs.append(pl.BlockSpec((dout, dnext), lambda i: (0, 0)))
        args.append(w_next)
    return pl.pallas_call(
        functools.partial(_hyp_body, relu=relu, has_w=has_w, concat=concat),
        grid=(n // bn,),
        in_specs=in_specs,
        out_specs=pl.BlockSpec((bn, dnext), lambda i: (i, 0)),
        out_shape=jax.ShapeDtypeStruct((n, dnext), jnp.float32),
    )(*args)


def _seg_kernel_body(dout, nchunks, rows_per_tile, col_split,
                     h_hbm, e_hbm, out_hbm,
                     src_v, dst_v, w_v, rows_v, h_sh, agg_sh, gsem, ssem):
    cid = lax.axis_index("c")
    sid = lax.axis_index("s")
    wid = sid if col_split else cid * NSUB + sid

    hrows = h_hbm.shape[-2] // NSUB
    row0 = sid * rows_per_tile
    d1 = pltpu.async_copy(e_hbm.at[0, wid], src_v, gsem.at[0])
    d2 = pltpu.async_copy(e_hbm.at[1, wid], dst_v, gsem.at[1])
    d3 = pltpu.async_copy(e_hbm.at[2, wid], w_v, gsem.at[2])
    if col_split:
        d4 = pltpu.async_copy(h_hbm.at[cid, pl.ds(sid * hrows, hrows)],
                              h_sh.at[pl.ds(sid * hrows, hrows)], gsem.at[3])
    else:
        d4 = pltpu.async_copy(h_hbm.at[pl.ds(sid * hrows, hrows)],
                              h_sh.at[pl.ds(sid * hrows, hrows)], gsem.at[3])

    def zrow(r, c2):
        for db in range(dout // 16):
            rows_v[0, r, pl.ds(db * 16, 16)] = jnp.zeros((16,), jnp.float32)
        return c2

    lax.fori_loop(0, CH, zrow, 0, unroll=8)
    zd = []
    for k in range(rows_per_tile // CH):
        zd.append(pltpu.async_copy(
            rows_v.at[0], agg_sh.at[pl.ds(row0 + k * CH, CH)],
            ssem.at[k % NBUF]))
    rem = rows_per_tile % CH
    if rem:
        zd.append(pltpu.async_copy(
            rows_v.at[0, pl.ds(0, rem)],
            agg_sh.at[pl.ds(row0 + (rows_per_tile // CH) * CH, rem)],
            ssem.at[rows_per_tile // CH % NBUF]))
    d1.wait()
    d2.wait()
    d3.wait()
    d4.wait()
    for d in zd:
        d.wait()
    plsc.subcore_barrier()

    def g_start(b, j):
        pltpu.async_copy(h_sh.at[src_v.at[j]], rows_v.at[b], gsem.at[b])

    def g_wait(b):
        pltpu.make_async_copy(h_sh.at[src_v.at[0]], rows_v.at[b],
                              gsem.at[b]).wait()

    def s_start(b, j):
        pltpu.async_copy(rows_v.at[b], agg_sh.at[dst_v.at[j]], ssem.at[b],
                         add=True)

    def s_wait(b):
        pltpu.make_async_copy(rows_v.at[b], agg_sh.at[dst_v.at[0]],
                              ssem.at[b]).wait()

    def scale(b, j):
        for g in range(CH // 16):
            wvec = lax.bitcast_convert_type(w_v[j, pl.ds(g * 16, 16)],
                                            jnp.float32)

            def edge_body(e16, c2, g=g, wvec=wvec):
                wv = lax.gather(
                    wvec, jnp.full((16, 1), 0, jnp.int32) + e16,
                    lax.GatherDimensionNumbers(
                        offset_dims=(), collapsed_slice_dims=(0,),
                        start_index_map=(0,)),
                    slice_sizes=(1,),
                    mode=lax.GatherScatterMode.PROMISE_IN_BOUNDS)
                e = g * 16 + e16
                for db in range(dout // 16):
                    seg = rows_v[b, e, pl.ds(db * 16, 16)]
                    rows_v[b, e, pl.ds(db * 16, 16)] = seg * wv
                return c2

            lax.fori_loop(0, 16, edge_body, 0, unroll=4)

    for jj in range(LOOK):
        g_start(jj, jj)

    def q_body(q, carry):
        for b in range(NBUF):
            j = q * NBUF + b
            g_wait(b)
            scale(b, j)
            bn = (b + LOOK) % NBUF

            @pl.when(j >= NBUF - LOOK)
            def _():
                s_wait(bn)

            @pl.when(j + LOOK < nchunks)
            def _():
                g_start(bn, j + LOOK)

            s_start(b, j)
        return carry

    lax.fori_loop(0, nchunks // NBUF, q_body, 0)
    for jj in range(nchunks - NBUF + LOOK, nchunks):
        s_wait(jj % NBUF)
    plsc.subcore_barrier()
    pltpu.sync_copy(agg_sh.at[pl.ds(row0, rows_per_tile)],
                    out_hbm.at[cid, pl.ds(row0, rows_per_tile)])


def _seg(h, e3, n, col_split=False):
    dout = h.shape[-1]
    nchunks = e3.shape[2]
    rows_per_tile = n // NSUB
    mesh = plsc.VectorSubcoreMesh(core_axis_name="c", subcore_axis_name="s")
    kfn = pl.kernel(
        functools.partial(_seg_kernel_body, dout, nchunks, rows_per_tile,
                          col_split),
        out_type=jax.ShapeDtypeStruct((NCORES, n, dout), jnp.float32),
        mesh=mesh,
        scratch_types=[
            pltpu.VMEM((nchunks, CH), jnp.int32),
            pltpu.VMEM((nchunks, CH), jnp.int32),
            pltpu.VMEM((nchunks, CH), jnp.int32),
            pltpu.VMEM((NBUF, CH, dout), jnp.float32),
            pltpu.VMEM_SHARED((h.shape[-2], dout), jnp.float32),
            pltpu.VMEM_SHARED((n, dout), jnp.float32),
            pltpu.SemaphoreType.DMA((NBUF,)),
            pltpu.SemaphoreType.DMA((NBUF,)),
        ],
        compiler_params=pltpu.CompilerParams(use_tc_tiling_on_sc=False),
    )
    return kfn(h, e3)


def kernel(x, edge_index, edge_weight, W0, b0, c0, W1, b1, c1, W2, b2, c2):
    n = x.shape[0]
    e = edge_weight.shape[0]
    per_tile = -(-e // NTILES)
    nchunks = -(-(-(-per_tile // CH)) // NBUF) * NBUF
    total = NTILES * nchunks * CH
    pad = total - e

    srcf = jnp.pad(edge_index[0].astype(jnp.int32), (0, pad))
    dstf = jnp.pad(edge_index[1].astype(jnp.int32), (0, pad))
    wf = lax.bitcast_convert_type(
        jnp.pad(edge_weight.astype(jnp.float32), (0, pad)), jnp.int32)
    ef = jnp.stack([srcf, dstf, wf])
    e3 = ef.reshape(3, NTILES, nchunks, CH)
    e2 = ef.reshape(3, NSUB, 2 * nchunks, CH)

    n_pad = -(-n // (NSUB * 8)) * (NSUB * 8)

    hs = _matmul_split(x, W0)
    parts = _seg(hs, e2, n_pad, col_split=True)
    h = _hyp(parts, b0, c0, W1, relu=True, concat=True)
    parts = _seg(h, e3, n_pad)
    h = _hyp(parts, b1, c1, W2, relu=True)
    parts = _seg(h, e3, n_pad)
    return _hyp(parts, b2, c2, None, relu=False)[:n]


# speedup vs baseline: 1.0241x; 1.0174x over previous
import functools

import jax
import jax.numpy as jnp
from jax import lax
from jax.experimental import pallas as pl
from jax.experimental.pallas import tpu as pltpu
from jax.experimental.pallas import tpu_sc as plsc

EPS = 1e-7

NCORES = 2
NSUB = 16
NTILES = NCORES * NSUB
CH = 128
NBUF = 4
LOOK = 2


def _mm_body(x_ref, w_ref, o_ref):
    res = jnp.dot(x_ref[...], w_ref[...],
                  preferred_element_type=jnp.float32)
    half = res.shape[1] // 2
    o_ref[0] = res[:, :half]
    o_ref[1] = res[:, half:]


def _matmul_split(x, w):
    n, din = x.shape
    dout = w.shape[1]
    bn = 1000
    return pl.pallas_call(
        _mm_body,
        grid=(n // bn,),
        in_specs=[
            pl.BlockSpec((bn, din), lambda i: (i, 0)),
            pl.BlockSpec((din, dout), lambda i: (0, 0)),
        ],
        out_specs=pl.BlockSpec((2, bn, dout // 2), lambda i: (0, i, 0)),
        out_shape=jax.ShapeDtypeStruct((2, n, dout // 2), jnp.float32),
    )(x, w)


def _rnorm(u):
    return jnp.maximum(jnp.sqrt(jnp.sum(u * u, axis=-1, keepdims=True)), EPS)


def _expmap0(u, sc):
    n = _rnorm(u)
    return jnp.tanh(sc * n) * u / (sc * n)


def _proj(p, sc):
    maxnorm = (1.0 - 1e-5) / sc
    n = _rnorm(p)
    return jnp.where(n > maxnorm, p / n * maxnorm, p)


def _logmap0(p, sc):
    n = _rnorm(p)
    arg = jnp.minimum(sc * n, 1.0 - 1e-5)
    atanh = 0.5 * jnp.log((1.0 + arg) / (1.0 - arg))
    return atanh * p / (sc * n)


def _mobius_add(x, y, c):
    x2 = jnp.sum(x * x, -1, keepdims=True)
    y2 = jnp.sum(y * y, -1, keepdims=True)
    xy = jnp.sum(x * y, -1, keepdims=True)
    num = (1 + 2 * c * xy + c * y2) * x + (1 - c * x2) * y
    den = 1 + 2 * c * xy + c * c * x2 * y2
    return num / jnp.maximum(den, EPS)


def _hyp_body(parts_ref, b_ref, c_ref, *rest, relu, has_w, concat):
    if has_w:
        w_ref, o_ref = rest
    else:
        (o_ref,) = rest
    c = jnp.maximum(c_ref[0], 1e-4)
    sc = jnp.sqrt(c)
    if concat:
        agg = jnp.concatenate([parts_ref[0], parts_ref[1]], axis=-1)
    else:
        agg = parts_ref[0] + parts_ref[1]
    p = _proj(_expmap0(agg, sc), sc)
    bh = _proj(_expmap0(b_ref[...], sc), sc)
    p = _proj(_mobius_add(p, bh, c), sc)
    out = _logmap0(p, sc)
    if relu:
        out = jnp.maximum(out, 0.0)
    if has_w:
        o_ref[...] = jnp.dot(out, w_ref[...],
                             preferred_element_type=jnp.float32)
    else:
        o_ref[...] = out


def _hyp(parts, b, c, w_next, relu, concat=False):
    _, n, dpart = parts.shape
    dout = 2 * dpart if concat else dpart
    bn = n // 8
    has_w = w_next is not None
    dnext = w_next.shape[1] if has_w else dout
    in_specs = [
        pl.BlockSpec((2, bn, dpart), lambda i: (0, i, 0)),
        pl.BlockSpec((1, dout), lambda i: (0, 0)),
        pl.BlockSpec(memory_space=pltpu.SMEM),
    ]
    args = [parts, b.reshape(1, dout), jnp.reshape(c, (1,))]
    if has_w:
        in_specs.append(pl.BlockSpec((dout, dnext), lambda i: (0, 0)))
        args.append(w_next)
    return pl.pallas_call(
        functools.partial(_hyp_body, relu=relu, has_w=has_w, concat=concat),
        grid=(n // bn,),
        in_specs=in_specs,
        out_specs=pl.BlockSpec((bn, dnext), lambda i: (i, 0)),
        out_shape=jax.ShapeDtypeStruct((n, dnext), jnp.float32),
    )(*args)


def _seg_kernel_body(dout, nchunks, rows_per_tile, col_split,
                     h_hbm, e_hbm, out_hbm,
                     src_v, dst_v, w_v, rows_v, h_sh, agg_sh, gsem, ssem):
    cid = lax.axis_index("c")
    sid = lax.axis_index("s")
    wid = sid if col_split else cid * NSUB + sid

    hrows = h_hbm.shape[-2] // NSUB
    row0 = sid * rows_per_tile
    d1 = pltpu.async_copy(e_hbm.at[0, wid], src_v, gsem.at[0])
    d2 = pltpu.async_copy(e_hbm.at[1, wid], dst_v, gsem.at[1])
    d3 = pltpu.async_copy(e_hbm.at[2, wid], w_v, gsem.at[2])
    if col_split:
        d4 = pltpu.async_copy(h_hbm.at[cid, pl.ds(sid * hrows, hrows)],
                              h_sh.at[pl.ds(sid * hrows, hrows)], gsem.at[3])
    else:
        d4 = pltpu.async_copy(h_hbm.at[pl.ds(sid * hrows, hrows)],
                              h_sh.at[pl.ds(sid * hrows, hrows)], gsem.at[3])

    def zrow(r, c2):
        for db in range(dout // 16):
            rows_v[0, r, pl.ds(db * 16, 16)] = jnp.zeros((16,), jnp.float32)
        return c2

    lax.fori_loop(0, CH, zrow, 0, unroll=8)
    zd = []
    for k in range(rows_per_tile // CH):
        zd.append(pltpu.async_copy(
            rows_v.at[0], agg_sh.at[pl.ds(row0 + k * CH, CH)],
            ssem.at[k % NBUF]))
    rem = rows_per_tile % CH
    if rem:
        zd.append(pltpu.async_copy(
            rows_v.at[0, pl.ds(0, rem)],
            agg_sh.at[pl.ds(row0 + (rows_per_tile // CH) * CH, rem)],
            ssem.at[rows_per_tile // CH % NBUF]))
    d1.wait()
    d2.wait()
    d3.wait()
    d4.wait()
    for d in zd:
        d.wait()
    plsc.subcore_barrier()

    def g_start(b, j):
        pltpu.async_copy(h_sh.at[src_v.at[j]], rows_v.at[b], gsem.at[b])

    def g_wait(b):
        pltpu.make_async_copy(h_sh.at[src_v.at[0]], rows_v.at[b],
                              gsem.at[b]).wait()

    def s_start(b, j):
        pltpu.async_copy(rows_v.at[b], agg_sh.at[dst_v.at[j]], ssem.at[b],
                         add=True)

    def s_wait(b):
        pltpu.make_async_copy(rows_v.at[b], agg_sh.at[dst_v.at[0]],
                              ssem.at[b]).wait()

    def scale(b, j):
        for g in range(CH // 16):
            wvec = lax.bitcast_convert_type(w_v[j, pl.ds(g * 16, 16)],
                                            jnp.float32)

            def edge_body(e16, c2, g=g, wvec=wvec):
                wv = lax.gather(
                    wvec, jnp.full((16, 1), 0, jnp.int32) + e16,
                    lax.GatherDimensionNumbers(
                        offset_dims=(), collapsed_slice_dims=(0,),
                        start_index_map=(0,)),
                    slice_sizes=(1,),
                    mode=lax.GatherScatterMode.PROMISE_IN_BOUNDS)
                e = g * 16 + e16
                for db in range(dout // 16):
                    seg = rows_v[b, e, pl.ds(db * 16, 16)]
                    rows_v[b, e, pl.ds(db * 16, 16)] = seg * wv
                return c2

            lax.fori_loop(0, 16, edge_body, 0, unroll=4)

    for jj in range(LOOK):
        g_start(jj, jj)

    def q_body(q, carry):
        for b in range(NBUF):
            j = q * NBUF + b
            g_wait(b)
            bn = (b + LOOK) % NBUF

            @pl.when(j >= NBUF - LOOK)
            def _():
                s_wait(bn)

            @pl.when(j + LOOK < nchunks)
            def _():
                g_start(bn, j + LOOK)

            scale(b, j)
            s_start(b, j)
        return carry

    lax.fori_loop(0, nchunks // NBUF, q_body, 0)
    for jj in range(nchunks - NBUF + LOOK, nchunks):
        s_wait(jj % NBUF)
    plsc.subcore_barrier()
    pltpu.sync_copy(agg_sh.at[pl.ds(row0, rows_per_tile)],
                    out_hbm.at[cid, pl.ds(row0, rows_per_tile)])


def _seg(h, e3, n, col_split=False):
    dout = h.shape[-1]
    nchunks = e3.shape[2]
    rows_per_tile = n // NSUB
    mesh = plsc.VectorSubcoreMesh(core_axis_name="c", subcore_axis_name="s")
    kfn = pl.kernel(
        functools.partial(_seg_kernel_body, dout, nchunks, rows_per_tile,
                          col_split),
        out_type=jax.ShapeDtypeStruct((NCORES, n, dout), jnp.float32),
        mesh=mesh,
        scratch_types=[
            pltpu.VMEM((nchunks, CH), jnp.int32),
            pltpu.VMEM((nchunks, CH), jnp.int32),
            pltpu.VMEM((nchunks, CH), jnp.int32),
            pltpu.VMEM((NBUF, CH, dout), jnp.float32),
            pltpu.VMEM_SHARED((h.shape[-2], dout), jnp.float32),
            pltpu.VMEM_SHARED((n, dout), jnp.float32),
            pltpu.SemaphoreType.DMA((NBUF,)),
            pltpu.SemaphoreType.DMA((NBUF,)),
        ],
        compiler_params=pltpu.CompilerParams(use_tc_tiling_on_sc=False),
    )
    return kfn(h, e3)


def kernel(x, edge_index, edge_weight, W0, b0, c0, W1, b1, c1, W2, b2, c2):
    n = x.shape[0]
    e = edge_weight.shape[0]
    per_tile = -(-e // NTILES)
    nchunks = -(-(-(-per_tile // CH)) // NBUF) * NBUF
    total = NTILES * nchunks * CH
    pad = total - e

    srcf = jnp.pad(edge_index[0].astype(jnp.int32), (0, pad))
    dstf = jnp.pad(edge_index[1].astype(jnp.int32), (0, pad))
    wf = lax.bitcast_convert_type(
        jnp.pad(edge_weight.astype(jnp.float32), (0, pad)), jnp.int32)
    ef = jnp.stack([srcf, dstf, wf])
    e3 = ef.reshape(3, NTILES, nchunks, CH)
    e2 = ef.reshape(3, NSUB, 2 * nchunks, CH)

    n_pad = -(-n // (NSUB * 8)) * (NSUB * 8)

    hs = _matmul_split(x, W0)
    parts = _seg(hs, e2, n_pad, col_split=True)
    h = _hyp(parts, b0, c0, W1, relu=True, concat=True)
    parts = _seg(h, e3, n_pad)
    h = _hyp(parts, b1, c1, W2, relu=True)
    parts = _seg(h, e3, n_pad)
    return _hyp(parts, b2, c2, None, relu=False)[:n]
